# baseline (speedup 1.0000x reference)
import functools

import jax
import jax.numpy as jnp
from jax import lax
from jax.experimental import pallas as pl
from jax.experimental.pallas import tpu as pltpu
from jax.experimental.pallas import tpu_sc as plsc

_NC = 2
_NS = 16
_NW = _NC * _NS

_V = 1000000
_D = 32
_CB = 128
_NB = _V // _CB
_TAIL = _V - _NB * _CB
_FULL_STEPS = _NB // _NW

_H = 200
_B = 4096
_NBB = _B // 128
_DEPTH = 4


def _transpose_block(in_v, out_v, iota, n_rows):
    for q in range(2):
        cvec = iota + q * 16
        for bl in range(n_rows):
            out_v[bl, pl.ds(q * 16, 16)] = plsc.load_gather(
                in_v, [cvec, jnp.full((16,), bl, jnp.int32)]
            )


def _make_transpose():
    mesh = plsc.VectorSubcoreMesh(core_axis_name="c", subcore_axis_name="s")

    @functools.partial(
        pl.kernel,
        out_type=jax.ShapeDtypeStruct((_V, _D), jnp.float32),
        mesh=mesh,
        scratch_types=[
            pltpu.VMEM((_DEPTH, _D, _CB), jnp.float32),
            pltpu.VMEM((_DEPTH, _CB, _D), jnp.float32),
            [pltpu.SemaphoreType.DMA] * _DEPTH,
            [pltpu.SemaphoreType.DMA] * _DEPTH,
        ],
        compiler_params=pltpu.CompilerParams(
            use_tc_tiling_on_sc=False, needs_layout_passes=False
        ),
    )
    def transpose_kernel(et_hbm, t_hbm, in_v, out_v, isems, osems):
        wid = lax.axis_index("s") * _NC + lax.axis_index("c")
        iota = lax.iota(jnp.int32, 16)

        def blk_of(g):
            return g * _NW + wid

        def fire_in(g, slot):
            pltpu.async_copy(
                et_hbm.at[:, pl.ds(blk_of(g) * _CB, _CB)],
                in_v.at[slot],
                isems[slot],
            )

        for b in range(_DEPTH):
            fire_in(b, b)

        def step(i, carry):
            for b in range(_DEPTH):
                g = i * _DEPTH + b
                pltpu.make_async_copy(
                    et_hbm.at[:, pl.ds(blk_of(g) * _CB, _CB)],
                    in_v.at[b],
                    isems[b],
                ).wait()

                @pl.when(i > 0)
                def _():
                    pltpu.make_async_copy(
                        out_v.at[b],
                        t_hbm.at[pl.ds(blk_of(g - _DEPTH) * _CB, _CB)],
                        osems[b],
                    ).wait()

                _transpose_block(in_v.at[b], out_v.at[b], iota, _CB)

                pltpu.async_copy(
                    out_v.at[b],
                    t_hbm.at[pl.ds(blk_of(g) * _CB, _CB)],
                    osems[b],
                )

                @pl.when(g + _DEPTH < _FULL_STEPS)
                def _():
                    fire_in(g + _DEPTH, b)
            return carry

        lax.fori_loop(0, _FULL_STEPS // _DEPTH, step, 0)

        for b in range(_DEPTH):
            g = _FULL_STEPS - _DEPTH + b
            pltpu.make_async_copy(
                out_v.at[b],
                t_hbm.at[pl.ds(blk_of(g) * _CB, _CB)],
                osems[b],
            ).wait()

        @pl.when(wid < 4)
        def _():
            i0 = (_FULL_STEPS * _NW + wid) * _CB
            pltpu.sync_copy(et_hbm.at[:, pl.ds(i0, _CB)], in_v.at[0])
            _transpose_block(in_v.at[0], out_v.at[0], iota, _CB)
            pltpu.sync_copy(out_v.at[0], t_hbm.at[pl.ds(i0, _CB)])

        @pl.when(wid == 4)
        def _():
            i0 = _NB * _CB
            pltpu.sync_copy(
                et_hbm.at[:, pl.ds(i0, _TAIL)],
                in_v.at[0, :, pl.ds(0, _TAIL)],
            )
            _transpose_block(in_v.at[0], out_v.at[0], iota, _TAIL)
            pltpu.sync_copy(
                out_v.at[0, pl.ds(0, _TAIL)], t_hbm.at[pl.ds(i0, _TAIL)]
            )

    return transpose_kernel


def _make_gather():
    mesh = plsc.VectorSubcoreMesh(core_axis_name="c", subcore_axis_name="s")

    @functools.partial(
        pl.kernel,
        out_type=jax.ShapeDtypeStruct((_H, 4, _NBB, 8, 128), jnp.float32),
        mesh=mesh,
        scratch_types=[
            pltpu.VMEM((_H, 128), jnp.int32),
            pltpu.VMEM((_DEPTH, 128, _D), jnp.float32),
            pltpu.VMEM((_DEPTH, 4, 8, 128), jnp.float32),
            [pltpu.SemaphoreType.DMA] * _DEPTH,
            [pltpu.SemaphoreType.DMA] * _DEPTH,
        ],
        compiler_params=pltpu.CompilerParams(
            use_tc_tiling_on_sc=False, needs_layout_passes=False
        ),
    )
    def gather_kernel(xt_hbm, t_hbm, o5_hbm, idx_v, rows_v, tile_v, gsems, osems):
        wid = lax.axis_index("s") * _NC + lax.axis_index("c")
        iota = lax.iota(jnp.int32, 16)

        pltpu.sync_copy(xt_hbm.at[:, pl.ds(wid * 128, 128)], idx_v)

        def fire_gather(h, slot):
            pltpu.async_copy(
                t_hbm.at[idx_v.at[h]], rows_v.at[slot], gsems[slot]
            )

        for b in range(_DEPTH):
            fire_gather(b, b)

        def step(i, carry):
            for b in range(_DEPTH):
                h = i * _DEPTH + b
                pltpu.make_async_copy(
                    t_hbm.at[idx_v.at[h]], rows_v.at[b], gsems[b]
                ).wait()

                @pl.when(i > 0)
                def _():
                    for fb in range(4):
                        pltpu.make_async_copy(
                            tile_v.at[b, fb],
                            o5_hbm.at[h - _DEPTH, fb, wid],
                            osems[b],
                        ).wait()

                for fb in range(4):
                    for fl in range(8):
                        cvec = jnp.full((16,), fb * 8 + fl, jnp.int32)
                        for q in range(8):
                            tile_v[b, fb, fl, pl.ds(q * 16, 16)] = (
                                plsc.load_gather(
                                    rows_v.at[b], [iota + q * 16, cvec]
                                )
                            )

                for fb in range(4):
                    pltpu.async_copy(
                        tile_v.at[b, fb], o5_hbm.at[h, fb, wid], osems[b]
                    )

                @pl.when(h + _DEPTH < _H)
                def _():
                    fire_gather(h + _DEPTH, b)
            return carry

        lax.fori_loop(0, _H // _DEPTH, step, 0)

        for b in range(_DEPTH):
            h = _H - _DEPTH + b
            for fb in range(4):
                pltpu.make_async_copy(
                    tile_v.at[b, fb], o5_hbm.at[h, fb, wid], osems[b]
                ).wait()

    return gather_kernel


def kernel(x, embedding):
    et = embedding.T
    xt = x.T
    t2 = _make_transpose()(et)
    o5 = _make_gather()(xt, t2)
    return o5.transpose(2, 4, 0, 1, 3).reshape(_B, _H, _D)


# speedup vs baseline: 4.7070x; 4.7070x over previous
import functools

import jax
import jax.numpy as jnp
from jax import lax
from jax.experimental import pallas as pl
from jax.experimental.pallas import tpu as pltpu
from jax.experimental.pallas import tpu_sc as plsc

_NC = 2
_NS = 16
_NW = _NC * _NS

_V = 1000000
_D = 32
_H = 200
_B = 4096
_NBB = _B // 128
_DEPTH = 2


def _make_gather():
    mesh = plsc.VectorSubcoreMesh(core_axis_name="c", subcore_axis_name="s")

    @functools.partial(
        pl.kernel,
        out_type=jax.ShapeDtypeStruct((_H, 4, _NBB, 8, 128), jnp.float32),
        mesh=mesh,
        scratch_types=[
            pltpu.VMEM((_H, 128), jnp.int32),
            pltpu.VMEM((_DEPTH, 128, _D), jnp.float32),
            pltpu.VMEM((_DEPTH, 4, 8, 128), jnp.float32),
            [pltpu.SemaphoreType.DMA] * _DEPTH,
            [pltpu.SemaphoreType.DMA] * _DEPTH,
        ],
        compiler_params=pltpu.CompilerParams(
            use_tc_tiling_on_sc=False, needs_layout_passes=False
        ),
    )
    def gather_kernel(xt_hbm, t_hbm, o5_hbm, idx_v, rows_v, tile_v, gsems, osems):
        wid = lax.axis_index("s") * _NC + lax.axis_index("c")
        iota = lax.iota(jnp.int32, 16)

        dv = [(iota + k) & 15 for k in range(16)]
        flv = [d & 7 for d in dv]
        fhv = [d >> 3 for d in dv]
        blv = [iota + 16 * p for p in range(8)]

        pltpu.sync_copy(xt_hbm.at[:, pl.ds(wid * 128, 128)], idx_v)

        def fire_gather(h, slot):
            pltpu.async_copy(
                t_hbm.at[idx_v.at[h]], rows_v.at[slot], gsems[slot]
            )

        for b in range(_DEPTH):
            fire_gather(b, b)

        def step(i, carry):
            for b in range(_DEPTH):
                h = i * _DEPTH + b
                pltpu.make_async_copy(
                    t_hbm.at[idx_v.at[h]], rows_v.at[b], gsems[b]
                ).wait()

                @pl.when(i > 0)
                def _():
                    for fb in range(4):
                        pltpu.make_async_copy(
                            tile_v.at[b, fb],
                            o5_hbm.at[h - _DEPTH, fb, wid],
                            osems[b],
                        ).wait()

                for m in range(2):
                    for k in range(16):
                        col = dv[k] + 16 * m
                        fbv = fhv[k] + 2 * m
                        for p in range(8):
                            val = plsc.load_gather(
                                rows_v.at[b], [blv[p], col]
                            )
                            plsc.store_scatter(
                                tile_v.at[b], [fbv, flv[k], blv[p]], val
                            )

                for fb in range(4):
                    pltpu.async_copy(
                        tile_v.at[b, fb], o5_hbm.at[h, fb, wid], osems[b]
                    )

                @pl.when(h + _DEPTH < _H)
                def _():
                    fire_gather(h + _DEPTH, b)
            return carry

        lax.fori_loop(0, _H // _DEPTH, step, 0)

        for b in range(_DEPTH):
            h = _H - _DEPTH + b
            for fb in range(4):
                pltpu.make_async_copy(
                    tile_v.at[b, fb], o5_hbm.at[h, fb, wid], osems[b]
                ).wait()

    return gather_kernel


def kernel(x, embedding):
    xt = x.T
    o5 = _make_gather()(xt, embedding)
    return o5.transpose(2, 4, 0, 1, 3).reshape(_B, _H, _D)


# speedup vs baseline: 5.8295x; 1.2385x over previous
import functools

import jax
import jax.numpy as jnp
from jax import lax
from jax.experimental import pallas as pl
from jax.experimental.pallas import tpu as pltpu
from jax.experimental.pallas import tpu_sc as plsc

_NC = 2
_NS = 16
_NW = _NC * _NS

_V = 1000000
_D = 32
_H = 200
_B = 4096
_NBB = _B // 128
_DEPTH = 8


def _make_gather():
    mesh = plsc.VectorSubcoreMesh(core_axis_name="c", subcore_axis_name="s")

    @functools.partial(
        pl.kernel,
        out_type=jax.ShapeDtypeStruct((_H, 4, _NBB, 8, 128), jnp.float32),
        mesh=mesh,
        scratch_types=[
            pltpu.VMEM((_H, 128), jnp.int32),
            pltpu.VMEM((_DEPTH, 128, _D), jnp.float32),
            pltpu.VMEM((_DEPTH, 4, 8, 128), jnp.float32),
            [pltpu.SemaphoreType.DMA] * _DEPTH,
            [pltpu.SemaphoreType.DMA] * _DEPTH,
        ],
        compiler_params=pltpu.CompilerParams(
            use_tc_tiling_on_sc=False, needs_layout_passes=False
        ),
    )
    def gather_kernel(xt_hbm, t_hbm, o5_hbm, idx_v, rows_v, tile_v, gsems, osems):
        wid = lax.axis_index("s") * _NC + lax.axis_index("c")
        iota = lax.iota(jnp.int32, 16)
        blv = [iota + 16 * p for p in range(8)]

        pltpu.sync_copy(xt_hbm.at[:, pl.ds(wid * 128, 128)], idx_v)

        def fire_gather(h, slot):
            pltpu.async_copy(
                t_hbm.at[idx_v.at[h]], rows_v.at[slot], gsems[slot]
            )

        def retile(b):
            def dstep(k, carry):
                dv = (iota + k) & 15
                flv = dv & 7
                fhv = dv >> 3
                for m in range(2):
                    col = dv + 16 * m if m else dv
                    fbv = fhv + 2 * m if m else fhv
                    for p in range(8):
                        val = plsc.load_gather(rows_v.at[b], [blv[p], col])
                        plsc.store_scatter(
                            tile_v.at[b], [fbv, flv, blv[p]], val
                        )
                return carry

            lax.fori_loop(0, 16, dstep, 0)

        for b in range(_DEPTH):
            fire_gather(b, b)

        def step(i, carry):
            for b in range(_DEPTH):
                h = i * _DEPTH + b
                pltpu.make_async_copy(
                    t_hbm.at[idx_v.at[h]], rows_v.at[b], gsems[b]
                ).wait()

                @pl.when(i > 0)
                def _():
                    for fb in range(4):
                        pltpu.make_async_copy(
                            tile_v.at[b, fb],
                            o5_hbm.at[h - _DEPTH, fb, wid],
                            osems[b],
                        ).wait()

                retile(b)

                for fb in range(4):
                    pltpu.async_copy(
                        tile_v.at[b, fb], o5_hbm.at[h, fb, wid], osems[b]
                    )

                @pl.when(h + _DEPTH < _H)
                def _():
                    fire_gather(h + _DEPTH, b)
            return carry

        lax.fori_loop(0, _H // _DEPTH, step, 0)

        for b in range(_DEPTH):
            h = _H - _DEPTH + b
            for fb in range(4):
                pltpu.make_async_copy(
                    tile_v.at[b, fb], o5_hbm.at[h, fb, wid], osems[b]
                ).wait()

    return gather_kernel


def kernel(x, embedding):
    xt = x.T
    o5 = _make_gather()(xt, embedding)
    return o5.transpose(2, 4, 0, 1, 3).reshape(_B, _H, _D)


# speedup vs baseline: 8.4465x; 1.4489x over previous
import functools

import jax
import jax.numpy as jnp
from jax import lax
from jax.experimental import pallas as pl
from jax.experimental.pallas import tpu as pltpu
from jax.experimental.pallas import tpu_sc as plsc

_NC = 2
_NS = 16
_NW = _NC * _NS

_V = 1000000
_D = 32
_CB = 128
_NBLK = _V // _CB
_ASTEPS = _NBLK // _NW
_AEPI = _NBLK - _ASTEPS * _NW
_H = 200
_B = 4096
_NBB = _B // 128
_DEPTH = 8
_ADEPTH = 4


def _make_relayout():
    mesh = plsc.VectorSubcoreMesh(core_axis_name="c", subcore_axis_name="s")

    @functools.partial(
        pl.kernel,
        out_type=jax.ShapeDtypeStruct((_V * _D,), jnp.float32),
        mesh=mesh,
        scratch_types=[
            [pltpu.VMEM((_D, _CB), jnp.float32)] * _ADEPTH,
            [pltpu.VMEM((_CB * _D,), jnp.float32)] * _ADEPTH,
            pltpu.VMEM((64, _D), jnp.float32),
            [pltpu.SemaphoreType.DMA] * _ADEPTH,
            [pltpu.SemaphoreType.DMA] * _ADEPTH,
        ],
        compiler_params=pltpu.CompilerParams(
            use_tc_tiling_on_sc=True, needs_layout_passes=False
        ),
    )
    def relayout_kernel(et_hbm, tail_hbm, t_hbm, in_v, out_v, tail_v, isems, osems):
        wid = lax.axis_index("s") * _NC + lax.axis_index("c")
        iota = lax.iota(jnp.int32, 16)
        iota32 = iota * 32

        def blk_of(g):
            return g * _NW + wid

        def fire_in(g, slot):
            pltpu.async_copy(
                et_hbm.at[:, pl.ds(blk_of(g) * _CB, _CB)],
                in_v[slot],
                isems[slot],
            )

        def retile(slot):
            def dstep(k, carry):
                dv = (iota + k) & 15
                fl32 = iota32 + dv
                for m in range(2):
                    cvec = dv + 16 * m if m else dv
                    for p in range(8):
                        val = plsc.load_gather(
                            in_v[slot], [cvec, iota + 16 * p]
                        )
                        plsc.store_scatter(
                            out_v[slot], [fl32 + (512 * p + 16 * m)], val
                        )
                return carry

            lax.fori_loop(0, 16, dstep, 0)

        for b in range(_ADEPTH):
            fire_in(b, b)

        def step(i, carry):
            for b in range(_ADEPTH):
                g = i * _ADEPTH + b
                pltpu.make_async_copy(
                    et_hbm.at[:, pl.ds(blk_of(g) * _CB, _CB)],
                    in_v[b],
                    isems[b],
                ).wait()

                @pl.when(i > 0)
                def _():
                    pltpu.make_async_copy(
                        out_v[b],
                        t_hbm.at[pl.ds(blk_of(g - _ADEPTH) * _CB * _D, _CB * _D)],
                        osems[b],
                    ).wait()

                retile(b)

                pltpu.async_copy(
                    out_v[b],
                    t_hbm.at[pl.ds(blk_of(g) * _CB * _D, _CB * _D)],
                    osems[b],
                )

                @pl.when(g + _ADEPTH < _ASTEPS)
                def _():
                    fire_in(g + _ADEPTH, b)
            return carry

        lax.fori_loop(0, _ASTEPS // _ADEPTH, step, 0)

        for b in range(_ADEPTH):
            g = _ASTEPS - _ADEPTH + b
            pltpu.make_async_copy(
                out_v[b],
                t_hbm.at[pl.ds(blk_of(g) * _CB * _D, _CB * _D)],
                osems[b],
            ).wait()

        @pl.when(wid < _AEPI)
        def _():
            blk = _ASTEPS * _NW + wid
            pltpu.sync_copy(
                et_hbm.at[:, pl.ds(blk * _CB, _CB)], in_v[0]
            )
            retile(0)
            pltpu.sync_copy(
                out_v[0], t_hbm.at[pl.ds(blk * _CB * _D, _CB * _D)]
            )

        @pl.when(wid == _AEPI)
        def _():
            pltpu.sync_copy(tail_hbm, tail_v)
            for u in range(128):
                out_v[0][pl.ds(16 * u, 16)] = plsc.load_gather(
                    tail_v, [jnp.full((16,), u // 2, jnp.int32),
                             iota + 16 * (u % 2)]
                )
            pltpu.sync_copy(
                out_v[0].at[pl.ds(0, 2048)],
                t_hbm.at[pl.ds(_NBLK * _CB * _D, 2048)],
            )

    return relayout_kernel


def _make_gather():
    mesh = plsc.VectorSubcoreMesh(core_axis_name="c", subcore_axis_name="s")

    @functools.partial(
        pl.kernel,
        out_type=jax.ShapeDtypeStruct((_H, 4, _NBB, 8, 128), jnp.float32),
        mesh=mesh,
        scratch_types=[
            pltpu.VMEM((_H, 128), jnp.int32),
            pltpu.VMEM((_DEPTH, 128, _D), jnp.float32),
            pltpu.VMEM((_DEPTH, 4, 8, 128), jnp.float32),
            [pltpu.SemaphoreType.DMA] * _DEPTH,
            [pltpu.SemaphoreType.DMA] * _DEPTH,
        ],
        compiler_params=pltpu.CompilerParams(
            use_tc_tiling_on_sc=False, needs_layout_passes=False
        ),
    )
    def gather_kernel(xt_hbm, t_hbm, o5_hbm, idx_v, rows_v, tile_v, gsems, osems):
        wid = lax.axis_index("s") * _NC + lax.axis_index("c")
        iota = lax.iota(jnp.int32, 16)
        blv = [iota + 16 * p for p in range(8)]

        pltpu.sync_copy(xt_hbm.at[:, pl.ds(wid * 128, 128)], idx_v)

        def fire_gather(h, slot):
            pltpu.async_copy(
                t_hbm.at[idx_v.at[h]], rows_v.at[slot], gsems[slot]
            )

        def retile(b):
            def dstep(k, carry):
                dv = (iota + k) & 15
                flv = dv & 7
                fhv = dv >> 3
                for m in range(2):
                    col = dv + 16 * m if m else dv
                    fbv = fhv + 2 * m if m else fhv
                    for p in range(8):
                        val = plsc.load_gather(rows_v.at[b], [blv[p], col])
                        plsc.store_scatter(
                            tile_v.at[b], [fbv, flv, blv[p]], val
                        )
                return carry

            lax.fori_loop(0, 16, dstep, 0)

        for b in range(_DEPTH):
            fire_gather(b, b)

        def step(i, carry):
            for b in range(_DEPTH):
                h = i * _DEPTH + b
                pltpu.make_async_copy(
                    t_hbm.at[idx_v.at[h]], rows_v.at[b], gsems[b]
                ).wait()

                @pl.when(i > 0)
                def _():
                    for fb in range(4):
                        pltpu.make_async_copy(
                            tile_v.at[b, fb],
                            o5_hbm.at[h - _DEPTH, fb, wid],
                            osems[b],
                        ).wait()

                retile(b)

                for fb in range(4):
                    pltpu.async_copy(
                        tile_v.at[b, fb], o5_hbm.at[h, fb, wid], osems[b]
                    )

                @pl.when(h + _DEPTH < _H)
                def _():
                    fire_gather(h + _DEPTH, b)
            return carry

        lax.fori_loop(0, _H // _DEPTH, step, 0)

        for b in range(_DEPTH):
            h = _H - _DEPTH + b
            for fb in range(4):
                pltpu.make_async_copy(
                    tile_v.at[b, fb], o5_hbm.at[h, fb, wid], osems[b]
                ).wait()

    return gather_kernel


def kernel(x, embedding):
    xt = x.T
    et = embedding.T
    e_tail = embedding[_NBLK * _CB:]
    t1 = _make_relayout()(et, e_tail)
    t2 = t1.reshape(_V, _D)
    o5 = _make_gather()(xt, t2)
    return o5.transpose(2, 4, 0, 1, 3).reshape(_B, _H, _D)


# speedup vs baseline: 20.9755x; 2.4834x over previous
import functools

import jax
import jax.numpy as jnp
from jax import lax
from jax.experimental import pallas as pl
from jax.experimental.pallas import tpu as pltpu
from jax.experimental.pallas import tpu_sc as plsc

_NC = 2
_NS = 16
_NW = _NC * _NS

_V = 1000000
_D = 32
_CB = 128
_NBLK = _V // _CB
_ASTEPS = _NBLK // _NW
_AEPI = _NBLK - _ASTEPS * _NW
_H = 200
_B = 4096
_NBB = _B // 128
_DEPTH = 8
_ADEPTH = 4


def _make_relayout():
    mesh = plsc.VectorSubcoreMesh(core_axis_name="c", subcore_axis_name="s")

    @functools.partial(
        pl.kernel,
        out_type=jax.ShapeDtypeStruct((_V * _D,), jnp.float32),
        mesh=mesh,
        scratch_types=[
            [pltpu.VMEM((_D, _CB), jnp.float32)] * _ADEPTH,
            [pltpu.VMEM((_CB * _D,), jnp.float32)] * _ADEPTH,
            pltpu.VMEM((64, _D), jnp.float32),
            [pltpu.SemaphoreType.DMA] * _ADEPTH,
            [pltpu.SemaphoreType.DMA] * _ADEPTH,
        ],
        compiler_params=pltpu.CompilerParams(
            use_tc_tiling_on_sc=True, needs_layout_passes=False
        ),
    )
    def relayout_kernel(et_hbm, tail_hbm, t_hbm, in_v, out_v, tail_v, isems, osems):
        wid = lax.axis_index("s") * _NC + lax.axis_index("c")
        iota = lax.iota(jnp.int32, 16)
        iota32 = iota * 32

        def blk_of(g):
            return g * _NW + wid

        def fire_in(g, slot):
            pltpu.async_copy(
                et_hbm.at[:, pl.ds(blk_of(g) * _CB, _CB)],
                in_v[slot],
                isems[slot],
            )

        def retile(slot):
            def dstep(k, carry):
                dv = (iota + k) & 15
                fl32 = iota32 + dv
                for m in range(2):
                    cvec = dv + 16 * m if m else dv
                    vals = [
                        plsc.load_gather(in_v[slot], [cvec, iota + 16 * p])
                        for p in range(8)
                    ]
                    for p in range(8):
                        plsc.store_scatter(
                            out_v[slot], [fl32 + (512 * p + 16 * m)], vals[p]
                        )
                return carry

            lax.fori_loop(0, 16, dstep, 0)

        for b in range(_ADEPTH):
            fire_in(b, b)

        def step(i, carry):
            for b in range(_ADEPTH):
                g = i * _ADEPTH + b
                pltpu.make_async_copy(
                    et_hbm.at[:, pl.ds(blk_of(g) * _CB, _CB)],
                    in_v[b],
                    isems[b],
                ).wait()

                @pl.when(i > 0)
                def _():
                    pltpu.make_async_copy(
                        out_v[b],
                        t_hbm.at[pl.ds(blk_of(g - _ADEPTH) * _CB * _D, _CB * _D)],
                        osems[b],
                    ).wait()

                retile(b)

                pltpu.async_copy(
                    out_v[b],
                    t_hbm.at[pl.ds(blk_of(g) * _CB * _D, _CB * _D)],
                    osems[b],
                )

                @pl.when(g + _ADEPTH < _ASTEPS)
                def _():
                    fire_in(g + _ADEPTH, b)
            return carry

        lax.fori_loop(0, _ASTEPS // _ADEPTH, step, 0)

        for b in range(_ADEPTH):
            g = _ASTEPS - _ADEPTH + b
            pltpu.make_async_copy(
                out_v[b],
                t_hbm.at[pl.ds(blk_of(g) * _CB * _D, _CB * _D)],
                osems[b],
            ).wait()

        @pl.when(wid < _AEPI)
        def _():
            blk = _ASTEPS * _NW + wid
            pltpu.sync_copy(
                et_hbm.at[:, pl.ds(blk * _CB, _CB)], in_v[0]
            )
            retile(0)
            pltpu.sync_copy(
                out_v[0], t_hbm.at[pl.ds(blk * _CB * _D, _CB * _D)]
            )

        @pl.when(wid == _AEPI)
        def _():
            pltpu.sync_copy(tail_hbm, tail_v)
            for u in range(128):
                out_v[0][pl.ds(16 * u, 16)] = plsc.load_gather(
                    tail_v, [jnp.full((16,), u // 2, jnp.int32),
                             iota + 16 * (u % 2)]
                )
            pltpu.sync_copy(
                out_v[0].at[pl.ds(0, 2048)],
                t_hbm.at[pl.ds(_NBLK * _CB * _D, 2048)],
            )

    return relayout_kernel


def _make_gather():
    mesh = plsc.VectorSubcoreMesh(core_axis_name="c", subcore_axis_name="s")

    @functools.partial(
        pl.kernel,
        out_type=jax.ShapeDtypeStruct((_H, 4, _NBB, 8, 128), jnp.float32),
        mesh=mesh,
        scratch_types=[
            pltpu.VMEM((_H, 128), jnp.int32),
            pltpu.VMEM((_DEPTH, 128, _D), jnp.float32),
            pltpu.VMEM((_DEPTH, 4, 8, 128), jnp.float32),
            [pltpu.SemaphoreType.DMA] * _DEPTH,
            [pltpu.SemaphoreType.DMA] * _DEPTH,
        ],
        compiler_params=pltpu.CompilerParams(
            use_tc_tiling_on_sc=False, needs_layout_passes=False
        ),
    )
    def gather_kernel(xt_hbm, t_hbm, o5_hbm, idx_v, rows_v, tile_v, gsems, osems):
        wid = lax.axis_index("s") * _NC + lax.axis_index("c")
        iota = lax.iota(jnp.int32, 16)
        blv = [iota + 16 * p for p in range(8)]

        pltpu.sync_copy(xt_hbm.at[:, pl.ds(wid * 128, 128)], idx_v)

        def fire_gather(h, slot):
            pltpu.async_copy(
                t_hbm.at[idx_v.at[h]], rows_v.at[slot], gsems[slot]
            )

        def retile(b):
            def dstep(k, carry):
                dv = (iota + k) & 15
                flv = dv & 7
                fhv = dv >> 3
                for m in range(2):
                    col = dv + 16 * m if m else dv
                    fbv = fhv + 2 * m if m else fhv
                    vals = [
                        plsc.load_gather(rows_v.at[b], [blv[p], col])
                        for p in range(8)
                    ]
                    for p in range(8):
                        plsc.store_scatter(
                            tile_v.at[b], [fbv, flv, blv[p]], vals[p]
                        )
                return carry

            lax.fori_loop(0, 16, dstep, 0)

        for b in range(_DEPTH):
            fire_gather(b, b)

        def step(i, carry):
            for b in range(_DEPTH):
                h = i * _DEPTH + b
                pltpu.make_async_copy(
                    t_hbm.at[idx_v.at[h]], rows_v.at[b], gsems[b]
                ).wait()

                @pl.when(i > 0)
                def _():
                    for fb in range(4):
                        pltpu.make_async_copy(
                            tile_v.at[b, fb],
                            o5_hbm.at[h - _DEPTH, fb, wid],
                            osems[b],
                        ).wait()

                retile(b)

                for fb in range(4):
                    pltpu.async_copy(
                        tile_v.at[b, fb], o5_hbm.at[h, fb, wid], osems[b]
                    )

                @pl.when(h + _DEPTH < _H)
                def _():
                    fire_gather(h + _DEPTH, b)
            return carry

        lax.fori_loop(0, _H // _DEPTH, step, 0)

        for b in range(_DEPTH):
            h = _H - _DEPTH + b
            for fb in range(4):
                pltpu.make_async_copy(
                    tile_v.at[b, fb], o5_hbm.at[h, fb, wid], osems[b]
                ).wait()

    return gather_kernel


def kernel(x, embedding):
    xt = x.T
    et = embedding.T
    e_tail = embedding[_NBLK * _CB:]
    t1 = _make_relayout()(et, e_tail)
    t2 = t1.reshape(_V, _D)
    o5 = _make_gather()(xt, t2)
    return o5.transpose(2, 4, 0, 1, 3).reshape(_B, _H, _D)
